# baseline (speedup 1.0000x reference)
import functools

import jax
import jax.numpy as jnp
from jax import lax
from jax.experimental import pallas as pl
from jax.experimental.pallas import tpu as pltpu
from jax.experimental.pallas import tpu_sc as plsc

VOC = 1_000_000
DIM = 32
NC = 2
NS = 16
NW = NC * NS


def _lookup_body(N, CHUNK, idx_hbm, w_hbm, out_hbm, idx_v, out_v, sem):
    wid = lax.axis_index("s") * NC + lax.axis_index("c")
    base = pl.multiple_of(wid * CHUNK, 8)
    pltpu.sync_copy(idx_hbm.at[pl.ds(base, CHUNK)], idx_v)

    def dbody(d, carry):
        pltpu.async_copy(w_hbm.at[idx_v], out_v, sem).wait()
        off = pl.multiple_of(d * N + base, 8)
        pltpu.sync_copy(out_v, out_hbm.at[pl.ds(off, CHUNK)])

        def addj(j, c):
            sl = pl.ds(pl.multiple_of(j * 16, 8), 16)
            idx_v[sl] = idx_v[sl] + VOC
            return c

        return lax.fori_loop(0, CHUNK // 16, addj, carry, unroll=8)

    lax.fori_loop(0, DIM, dbody, 0)


def kernel(input_data, weight):
    B, L = input_data.shape
    N = B * L
    CHUNK = N // NW
    assert N % (NW * 16) == 0

    idx_flat = input_data.reshape(N).astype(jnp.int32)
    w_flat = weight.reshape(DIM * VOC)

    run = functools.partial(
        pl.kernel,
        mesh=plsc.VectorSubcoreMesh(core_axis_name="c", subcore_axis_name="s"),
        out_type=jax.ShapeDtypeStruct((DIM * N,), jnp.float32),
        scratch_types=[
            pltpu.VMEM((CHUNK,), jnp.int32),
            pltpu.VMEM((CHUNK,), jnp.float32),
            pltpu.SemaphoreType.DMA,
        ],
    )(functools.partial(_lookup_body, N, CHUNK))

    out = run(idx_flat, w_flat)
    return out.reshape(DIM, B, L)


# speedup vs baseline: 3.2711x; 3.2711x over previous
import functools

import jax
import jax.numpy as jnp
from jax import lax
from jax.experimental import pallas as pl
from jax.experimental.pallas import tpu as pltpu
from jax.experimental.pallas import tpu_sc as plsc

VOC = 1_000_000
DIM = 32
NC = 2
NS = 16
NW = NC * NS

ALIGNED = 999_936
TAILW = VOC - ALIGNED
PITCH = VOC + TAILW
VCHUNK = 124_928
SUB = 7_680
N_SUB = 16
REM = VCHUNK - N_SUB * SUB
EXTRA = ALIGNED - 8 * VCHUNK


def _linearize_body(w_hbm, wtail_hbm, lin_hbm, blk_v, tail_v, sem):
    wid = lax.axis_index("s") * NC + lax.axis_index("c")
    g = wid // 8
    vc = wid % 8
    vbase = pl.multiple_of(vc * VCHUNK, 128)

    def stage(voff, width):
        pltpu.sync_copy(
            w_hbm.at[pl.ds(g * 8, 8), pl.ds(voff, width)],
            blk_v.at[:, pl.ds(0, width)],
        )
        for r in range(8):
            dst = pl.multiple_of((g * 8 + r) * PITCH, 8) + voff
            pltpu.sync_copy(
                blk_v.at[r, pl.ds(0, width)],
                lin_hbm.at[pl.ds(dst, width)],
            )

    def kbody(k, carry):
        stage(vbase + k * SUB, SUB)
        return carry

    lax.fori_loop(0, N_SUB, kbody, 0)
    stage(vbase + N_SUB * SUB, REM)

    @pl.when(vc == 7)
    def _extra():
        stage(8 * VCHUNK, EXTRA)

    @pl.when(wid == 31)
    def _tail():
        pltpu.sync_copy(wtail_hbm, tail_v)
        for d in range(DIM):
            pltpu.sync_copy(
                tail_v.at[d],
                lin_hbm.at[pl.ds(pl.multiple_of(d * PITCH + ALIGNED, 8), TAILW)],
            )


def _gather_body(N, CHUNK, idx_hbm, lin_hbm, out_hbm, idx_v, out_v, sem):
    wid = lax.axis_index("s") * NC + lax.axis_index("c")
    base = pl.multiple_of(wid * CHUNK, 8)
    pltpu.sync_copy(idx_hbm.at[pl.ds(base, CHUNK)], idx_v)

    def dbody(d, carry):
        pltpu.async_copy(lin_hbm.at[idx_v], out_v, sem).wait()
        off = pl.multiple_of(d * N + base, 8)
        pltpu.sync_copy(out_v, out_hbm.at[pl.ds(off, CHUNK)])

        def addj(j, c):
            sl = pl.ds(pl.multiple_of(j * 16, 8), 16)
            idx_v[sl] = idx_v[sl] + PITCH
            return c

        return lax.fori_loop(0, CHUNK // 16, addj, carry, unroll=8)

    lax.fori_loop(0, DIM, dbody, 0)


def kernel(input_data, weight):
    B, L = input_data.shape
    N = B * L
    CHUNK = N // NW
    assert N % (NW * 16) == 0

    idx = input_data.reshape(N).astype(jnp.int32)
    wtail = weight[:, ALIGNED:]

    mesh = plsc.VectorSubcoreMesh(core_axis_name="c", subcore_axis_name="s")

    linearize = functools.partial(
        pl.kernel,
        mesh=mesh,
        out_type=jax.ShapeDtypeStruct((DIM * PITCH,), jnp.float32),
        scratch_types=[
            pltpu.VMEM((8, SUB), jnp.float32),
            pltpu.VMEM((DIM, TAILW), jnp.float32),
            pltpu.SemaphoreType.DMA,
        ],
    )(_linearize_body)

    gather = functools.partial(
        pl.kernel,
        mesh=mesh,
        out_type=jax.ShapeDtypeStruct((DIM * N,), jnp.float32),
        scratch_types=[
            pltpu.VMEM((CHUNK,), jnp.int32),
            pltpu.VMEM((CHUNK,), jnp.float32),
            pltpu.SemaphoreType.DMA,
        ],
    )(functools.partial(_gather_body, N, CHUNK))

    w_lin = linearize(weight, wtail)
    out = gather(idx, w_lin)
    return out.reshape(DIM, B, L)
